# baseline (speedup 1.0000x reference)
import functools

import jax
import jax.numpy as jnp
from jax import lax
from jax.experimental import pallas as pl
from jax.experimental.pallas import tpu as pltpu
from jax.experimental.pallas import tpu_sc as plsc

VOCAB = 21
D = 128
VP = 32
B = 4096
L = 500
NLANES = 16
NW = 32
COLS_PER_W = B // NW
GROUPS = COLS_PER_W // NLANES

_mesh = plsc.VectorSubcoreMesh(core_axis_name="c", subcore_axis_name="s")


@functools.partial(
    pl.kernel,
    out_type=jax.ShapeDtypeStruct((B, VP), jnp.float32),
    mesh=_mesh,
    scratch_types=[
        pltpu.VMEM((L, COLS_PER_W), jnp.int32),
        pltpu.VMEM((COLS_PER_W, VP), jnp.float32),
    ],
    compiler_params=pltpu.CompilerParams(
        needs_layout_passes=False,
        use_tc_tiling_on_sc=False,
    ),
)
def _sc_hist(idxt_hbm, cnt_hbm, idx_v, cnt_v):
    wid = lax.axis_index("s") * 2 + lax.axis_index("c")
    base = wid * COLS_PER_W
    pltpu.sync_copy(idxt_hbm.at[:, pl.ds(base, COLS_PER_W)], idx_v)

    zf = jnp.zeros((NLANES,), jnp.float32)

    @pl.loop(0, COLS_PER_W)
    def _zero(c):
        cnt_v[c, pl.ds(0, NLANES)] = zf
        cnt_v[c, pl.ds(NLANES, NLANES)] = zf

    iota16 = lax.iota(jnp.int32, NLANES)
    ones = jnp.ones((NLANES,), jnp.float32)

    for g in range(GROUPS):
        cloc = iota16 + (g * NLANES)

        @plsc.parallel_loop(0, L, unroll=8)
        def _acc(l, g=g, cloc=cloc):
            ids = idx_v[l, pl.ds(g * NLANES, NLANES)]
            plsc.addupdate_scatter(cnt_v, [cloc, ids], ones)

    pltpu.sync_copy(cnt_v, cnt_hbm.at[pl.ds(base, COLS_PER_W)])


def _mm_body(cnt_ref, tab_ref, out_ref):
    out_ref[...] = lax.dot_general(
        cnt_ref[...], tab_ref[...],
        (((1,), (0,)), ((), ())),
        preferred_element_type=jnp.float32,
    ) * (1.0 / L)


_MM_BLK = 1024


def _tc_matmul(counts, tablep):
    return pl.pallas_call(
        _mm_body,
        grid=(B // _MM_BLK,),
        in_specs=[
            pl.BlockSpec((_MM_BLK, VP), lambda i: (i, 0)),
            pl.BlockSpec((VP, D), lambda i: (0, 0)),
        ],
        out_specs=pl.BlockSpec((_MM_BLK, D), lambda i: (i, 0)),
        out_shape=jax.ShapeDtypeStruct((B, D), jnp.float32),
    )(counts, tablep)


def kernel(indices, table):
    indices = indices.astype(jnp.int32)
    table = table.astype(jnp.float32)
    counts = _sc_hist(indices.T)
    tablep = jnp.concatenate(
        [table, jnp.zeros((VP - VOCAB, D), jnp.float32)], axis=0)
    return _tc_matmul(counts, tablep)


# speedup vs baseline: 1.5603x; 1.5603x over previous
import jax
import jax.numpy as jnp
from jax.experimental import pallas as pl

VOCAB = 21
EMBED_DIM = 128
PAD_IDX = 20

B_BLK = 512


def _body(idx_ref, tab_ref, out_ref):
    idx = idx_ref[...]
    acc = jnp.zeros((idx.shape[0], EMBED_DIM), jnp.float32)
    inv_l = 1.0 / idx.shape[1]
    for v in range(VOCAB):
        if v == PAD_IDX:
            continue
        cnt = jnp.sum((idx == v).astype(jnp.float32), axis=1, keepdims=True)
        acc = acc + cnt * tab_ref[v, :][None, :]
    out_ref[...] = acc * inv_l


def kernel(indices, table):
    indices = indices.astype(jnp.int32)
    table = table.astype(jnp.float32)
    b, l = indices.shape
    grid = (b // B_BLK,)
    return pl.pallas_call(
        _body,
        grid=grid,
        in_specs=[
            pl.BlockSpec((B_BLK, l), lambda i: (i, 0)),
            pl.BlockSpec((VOCAB, EMBED_DIM), lambda i: (0, 0)),
        ],
        out_specs=pl.BlockSpec((B_BLK, EMBED_DIM), lambda i: (i, 0)),
        out_shape=jax.ShapeDtypeStruct((b, EMBED_DIM), jnp.float32),
    )(indices, table)


# speedup vs baseline: 1.7238x; 1.1047x over previous
import jax
import jax.numpy as jnp
from jax.experimental import pallas as pl

VOCAB = 21
EMBED_DIM = 128
PAD_IDX = 20

B_BLK = 512


def _body(idxt_ref, tabt_ref, outt_ref):
    idx = idxt_ref[...]
    acc = jnp.zeros((EMBED_DIM, idx.shape[1]), jnp.float32)
    inv_l = 1.0 / idx.shape[0]
    for v in range(VOCAB):
        if v == PAD_IDX:
            continue
        cnt = jnp.sum((idx == v).astype(jnp.float32), axis=0, keepdims=True)
        acc = acc + tabt_ref[:, v][:, None] * cnt
    outt_ref[...] = acc * inv_l


def kernel(indices, table):
    indices = indices.astype(jnp.int32)
    table = table.astype(jnp.float32)
    b, l = indices.shape
    idxt = indices.T
    tabt = table.T
    outt = pl.pallas_call(
        _body,
        grid=(b // B_BLK,),
        in_specs=[
            pl.BlockSpec((l, B_BLK), lambda i: (0, i)),
            pl.BlockSpec((EMBED_DIM, VOCAB), lambda i: (0, 0)),
        ],
        out_specs=pl.BlockSpec((EMBED_DIM, B_BLK), lambda i: (0, i)),
        out_shape=jax.ShapeDtypeStruct((EMBED_DIM, b), jnp.float32),
    )(idxt, tabt)
    return outt.T


# speedup vs baseline: 1.9224x; 1.1152x over previous
import jax
import jax.numpy as jnp
from jax.experimental import pallas as pl

VOCAB = 21
EMBED_DIM = 128
PAD_IDX = 20

B_BLK = 1024


def _body(idxt_ref, tabt_ref, out_ref):
    idx = idxt_ref[...]
    acc = jnp.zeros((EMBED_DIM, idx.shape[1]), jnp.float32)
    inv_l = 1.0 / idx.shape[0]
    for v in range(VOCAB):
        if v == PAD_IDX:
            continue
        cnt = jnp.sum((idx == v).astype(jnp.float32), axis=0, keepdims=True)
        acc = acc + tabt_ref[:, v][:, None] * cnt
    out_ref[...] = acc.T * inv_l


def kernel(indices, table):
    indices = indices.astype(jnp.int32)
    table = table.astype(jnp.float32)
    b, l = indices.shape
    idxt = indices.T
    tabt = table.T
    return pl.pallas_call(
        _body,
        grid=(b // B_BLK,),
        in_specs=[
            pl.BlockSpec((l, B_BLK), lambda i: (0, i)),
            pl.BlockSpec((EMBED_DIM, VOCAB), lambda i: (0, 0)),
        ],
        out_specs=pl.BlockSpec((B_BLK, EMBED_DIM), lambda i: (i, 0)),
        out_shape=jax.ShapeDtypeStruct((b, EMBED_DIM), jnp.float32),
    )(idxt, tabt)


# speedup vs baseline: 1.9225x; 1.0000x over previous
import jax
import jax.numpy as jnp
from jax.experimental import pallas as pl

VOCAB = 21
EMBED_DIM = 128
PAD_IDX = 20

B_BLK = 2048


def _body(idxt_ref, tabt_ref, out_ref):
    idx = idxt_ref[...]
    acc = jnp.zeros((EMBED_DIM, idx.shape[1]), jnp.float32)
    inv_l = 1.0 / idx.shape[0]
    for v in range(VOCAB):
        if v == PAD_IDX:
            continue
        cnt = jnp.sum((idx == v).astype(jnp.float32), axis=0, keepdims=True)
        acc = acc + tabt_ref[:, v][:, None] * cnt
    out_ref[...] = acc.T * inv_l


def kernel(indices, table):
    indices = indices.astype(jnp.int32)
    table = table.astype(jnp.float32)
    b, l = indices.shape
    idxt = indices.T
    tabt = table.T
    return pl.pallas_call(
        _body,
        grid=(b // B_BLK,),
        in_specs=[
            pl.BlockSpec((l, B_BLK), lambda i: (0, i)),
            pl.BlockSpec((EMBED_DIM, VOCAB), lambda i: (0, 0)),
        ],
        out_specs=pl.BlockSpec((B_BLK, EMBED_DIM), lambda i: (i, 0)),
        out_shape=jax.ShapeDtypeStruct((b, EMBED_DIM), jnp.float32),
    )(idxt, tabt)


# speedup vs baseline: 1.9278x; 1.0028x over previous
import jax
import jax.numpy as jnp
from jax.experimental import pallas as pl

VOCAB = 21
EMBED_DIM = 128
PAD_IDX = 20

B_BLK = 1024


def _body(idxt_ref, tabt_ref, out_ref):
    idx = idxt_ref[...]
    acc = jnp.zeros((EMBED_DIM, idx.shape[1]), jnp.float32)
    inv_l = 1.0 / idx.shape[0]
    for v in range(VOCAB):
        if v == PAD_IDX:
            continue
        cnt = jnp.sum((idx == v).astype(jnp.float32), axis=0, keepdims=True)
        acc = acc + tabt_ref[:, v][:, None] * cnt
    out_ref[...] = acc.T * inv_l


def kernel(indices, table):
    indices = indices.astype(jnp.int32)
    table = table.astype(jnp.float32)
    b, l = indices.shape
    idxt = indices.T
    tabt = table.T
    return pl.pallas_call(
        _body,
        grid=(b // B_BLK,),
        in_specs=[
            pl.BlockSpec((l, B_BLK), lambda i: (0, i)),
            pl.BlockSpec((EMBED_DIM, VOCAB), lambda i: (0, 0)),
        ],
        out_specs=pl.BlockSpec((B_BLK, EMBED_DIM), lambda i: (i, 0)),
        out_shape=jax.ShapeDtypeStruct((b, EMBED_DIM), jnp.float32),
    )(idxt, tabt)
